# Initial kernel scaffold; baseline (speedup 1.0000x reference)
import jax
import jax.numpy as jnp
from jax.experimental import pallas as pl


def kernel(feat_w, bias_feat_w, train_x):
    raise NotImplementedError("write your pallas kernel here")


# speedup vs baseline: 1.3675x; 1.3675x over previous
import functools

import jax
import jax.numpy as jnp
from jax import lax
from jax.experimental import pallas as pl
from jax.experimental.pallas import tpu as pltpu
from jax.experimental.pallas import tpu_sc as plsc

N_FEAT = 1000000
K = 16
BATCH = 16384
N_FIELDS = 26

NC = 2
NS = 16
NW = NC * NS
ROWS_PER_W = BATCH // NW
BG = 128
NCHUNK = ROWS_PER_W // BG
NBUF = 2


def _mf_body(feat_hbm, bias_hbm, xt_hbm, out_hbm,
             rows_v, idx_v, bias_v, out_v,
             sem_in0, sem_in1, sem_out0, sem_out1):
    wid = lax.axis_index("s") * NC + lax.axis_index("c")
    wbase = wid * ROWS_PER_W
    sems_in = (sem_in0, sem_in1)
    sems_out = (sem_out0, sem_out1)

    def in_copies(buf):
        cs = []
        for f in range(N_FIELDS):
            cs.append(pltpu.make_async_copy(
                feat_hbm.at[idx_v.at[buf, f]], rows_v.at[buf, f], sems_in[buf]))
            cs.append(pltpu.make_async_copy(
                bias_hbm.at[idx_v.at[buf, f]], bias_v.at[buf, f], sems_in[buf]))
        return cs

    def fire(t, buf):
        base = wbase + t * BG
        pltpu.sync_copy(xt_hbm.at[:, pl.ds(base, BG)], idx_v.at[buf])
        for c in in_copies(buf):
            c.start()

    def drain(buf):
        for c in in_copies(buf):
            c.wait()

    def out_copy(t, buf):
        base = wbase + t * BG
        return pltpu.make_async_copy(
            out_v.at[buf], out_hbm.at[pl.ds(base, BG)], sems_out[buf])

    lane = lax.iota(jnp.int32, 16)
    bfly = [jnp.reshape(jnp.bitwise_xor(lane, 1 << p), (16, 1)) for p in range(4)]
    _dnums = lax.GatherDimensionNumbers(
        offset_dims=(), collapsed_slice_dims=(0,), start_index_map=(0,))

    def shuffle(x, idx2):
        return lax.gather(x, idx2, _dnums, slice_sizes=(1,),
                          mode=lax.GatherScatterMode.PROMISE_IN_BOUNDS)

    def compute(buf):
        zeros = jnp.zeros((16,), jnp.float32)

        def row_body(r, fmv):
            j = jnp.bitwise_and(r, 15)
            v0 = rows_v[buf, 0, r]
            s = v0
            q = v0 * v0
            for f in range(1, N_FIELDS):
                v = rows_v[buf, f, r]
                s = s + v
                q = q + v * v
            d = s * s - q
            for p in range(4):
                d = d + shuffle(d, bfly[p])
            fmv = jnp.where(lane == j, d, fmv)

            @pl.when(j == 15)
            def _():
                b0 = r - 15
                bacc = bias_v[buf, 0, pl.ds(b0, 16)]
                for f in range(1, N_FIELDS):
                    bacc = bacc + bias_v[buf, f, pl.ds(b0, 16)]
                out_v[buf, pl.ds(b0, 16)] = fmv * 0.5 + bacc

            return jnp.where(j == 15, zeros, fmv)

        lax.fori_loop(0, BG, row_body, zeros)

    fire(0, 0)
    fire(1, 1)
    for t in range(NCHUNK):
        buf = t % NBUF
        drain(buf)
        if t >= NBUF:
            out_copy(t - NBUF, buf).wait()
        compute(buf)
        out_copy(t, buf).start()
        nt = t + NBUF
        if nt < NCHUNK:
            fire(nt, buf)
    for t in range(max(NCHUNK - NBUF, 0), NCHUNK):
        out_copy(t, t % NBUF).wait()


_mf_call = functools.partial(
    pl.kernel,
    out_type=jax.ShapeDtypeStruct((BATCH,), jnp.float32),
    mesh=plsc.VectorSubcoreMesh(core_axis_name="c", subcore_axis_name="s"),
    compiler_params=pltpu.CompilerParams(use_tc_tiling_on_sc=False),
    scratch_types=[
        pltpu.VMEM((NBUF, N_FIELDS, BG, K), jnp.float32),
        pltpu.VMEM((NBUF, N_FIELDS, BG), jnp.int32),
        pltpu.VMEM((NBUF, N_FIELDS, BG), jnp.float32),
        pltpu.VMEM((NBUF, BG), jnp.float32),
        pltpu.SemaphoreType.DMA,
        pltpu.SemaphoreType.DMA,
        pltpu.SemaphoreType.DMA,
        pltpu.SemaphoreType.DMA,
    ],
)(_mf_body)


def kernel(feat_w, bias_feat_w, train_x):
    xt = jnp.transpose(train_x)
    bias_flat = jnp.reshape(bias_feat_w, (N_FEAT,))
    return _mf_call(feat_w, bias_flat, xt)
